# baseline (speedup 1.0000x reference)
import functools

import jax
import jax.numpy as jnp
from jax import lax
from jax.experimental import pallas as pl
from jax.experimental.pallas import tpu as pltpu
from jax.experimental.pallas import tpu_sc as plsc

N_NODES = 10000
N_PAD = 10240
N_EDGES = 320000
D_FEAT = 128
D_EDGE = 16
D_OUT = 64

NC = 2
NS = 16
EC = N_EDGES // (NC * NS)
SUB = 40
DSUB = 80


_DEG_BLK = EC // DSUB


def _deg_body(dst_hbm, ones_hbm, zeros_hbm, deg_out, deg_sp, dst_v, ones_v, sem):
    c = lax.axis_index("c")
    s = lax.axis_index("s")
    nrow = N_PAD // NS
    pltpu.sync_copy(zeros_hbm, deg_sp.at[pl.ds(s * nrow, nrow)])
    pltpu.sync_copy(ones_hbm, ones_v)
    pltpu.sync_copy(dst_hbm.at[c * NS + s], dst_v)
    plsc.subcore_barrier()

    def fire(j, _):
        pltpu.async_copy(ones_v, deg_sp.at[dst_v.at[j]], sem, add=True)
        return 0

    def drain(j, _):
        pltpu.make_async_copy(ones_v, deg_sp.at[dst_v.at[0]], sem).wait()
        return 0

    lax.fori_loop(0, _DEG_BLK, fire, 0)
    lax.fori_loop(0, _DEG_BLK, drain, 0)
    plsc.subcore_barrier()
    pltpu.sync_copy(deg_sp.at[pl.ds(s * nrow, nrow)],
                    deg_out.at[c, pl.ds(s * nrow, nrow)])


B_E = 40
_NB = EC // B_E
_NSUB = B_E // SUB


def _agg_body(src_hbm, dst_hbm, yd_hbm, z_hbm, zeros_hbm, acc_out,
              acc_sp, *bufs):
    c = lax.axis_index("c")
    s = lax.axis_index("s")
    nrow = N_PAD // NS
    pltpu.sync_copy(zeros_hbm, acc_sp.at[pl.ds(s * nrow, nrow)])
    plsc.subcore_barrier()
    w = c * NS + s
    S = [tuple(bufs[8 * p:8 * p + 8]) for p in range(4)]

    def fire_idx(i, p):
        src_v, dst_v = S[p][0], S[p][1]
        isem = S[p][4]
        pltpu.async_copy(src_hbm.at[w, i], src_v, isem)
        pltpu.async_copy(dst_hbm.at[w, i], dst_v, isem)

    def fire_gather(i, p):
        src_v, dst_v, yd_v, z_v, isem, gsem, zsem, _ = S[p]
        pltpu.make_async_copy(src_hbm.at[w, 0], src_v, isem).wait()
        pltpu.make_async_copy(dst_hbm.at[w, 0], dst_v, isem).wait()
        pltpu.async_copy(yd_hbm.at[src_v.at[0]], yd_v, gsem)
        pltpu.async_copy(z_hbm.at[pl.ds((w * _NB + i) * B_E, B_E)], z_v, zsem)

    def compute_scatter(p):
        src_v, dst_v, yd_v, z_v, isem, gsem, zsem, ssem = S[p]
        pltpu.make_async_copy(yd_hbm.at[src_v.at[0]], yd_v, gsem).wait()
        pltpu.make_async_copy(z_hbm.at[pl.ds(0, B_E)], z_v, zsem).wait()

        def group(r, _):
            db = yd_v[r, pl.ds(D_OUT, 16)]
            for seg in range(D_OUT // 16):
                yv = yd_v[r, pl.ds(seg * 16, 16)]
                zv = z_v[r, pl.ds(seg * 16, 16)]
                yd_v[r, pl.ds(seg * 16, 16)] = yv + db * zv
            return 0

        lax.fori_loop(0, B_E, group, 0)
        pltpu.async_copy(yd_v, acc_sp.at[dst_v.at[0]], ssem, add=True)

    def drain(p):
        dst_v, yd_v, ssem = S[p][1], S[p][2], S[p][7]
        pltpu.make_async_copy(yd_v, acc_sp.at[dst_v.at[0]], ssem).wait()

    fire_idx(0, 0)
    fire_idx(1, 1)
    fire_gather(0, 0)

    def quad(k, _):
        for j in range(4):
            i = 4 * k + j
            compute_scatter(j)

            @pl.when(i >= 1)
            def _():
                drain((j + 3) % 4)

            fire_idx(i + 2, (j + 2) % 4)
            fire_gather(i + 1, (j + 1) % 4)
        return 0

    lax.fori_loop(0, _NB // 4, quad, 0)
    compute_scatter(0)
    drain(3)
    fire_gather(249, 1)
    compute_scatter(1)
    drain(0)
    drain(1)
    plsc.subcore_barrier()
    pltpu.sync_copy(acc_sp.at[pl.ds(s * nrow, nrow)],
                    acc_out.at[c, pl.ds(s * nrow, nrow), :])


def _prep_body(deg0_ref, deg1_ref, x_ref, wx_ref, yd_ref):
    deg = deg0_ref[...] + deg1_ref[...] + 1.0
    dinv = deg ** -0.5
    y = jnp.dot(x_ref[...], wx_ref[...], preferred_element_type=jnp.float32)
    yd_ref[...] = jnp.concatenate(
        [dinv * y, jnp.broadcast_to(dinv, (dinv.shape[0], D_OUT))], axis=1)


def _edge_mm_body(ea_ref, we_ref, z_ref):
    z_ref[...] = jnp.dot(ea_ref[...], we_ref[...],
                         preferred_element_type=jnp.float32)


def _final_body(acc0_ref, acc1_ref, yd_ref, deg0_ref, deg1_ref,
                b_ref, out_ref):
    dinv = yd_ref[:, D_OUT:D_OUT + 1]
    ys = yd_ref[:, :D_OUT]
    acc = acc0_ref[:, :D_OUT] + acc1_ref[:, :D_OUT]
    deg = deg0_ref[...] + deg1_ref[...] + 1.0
    out_ref[...] = dinv * (acc + ys) + deg * b_ref[...]


@jax.jit
def kernel(x, edge_index, edge_attr, W, b):
    f32 = jnp.float32
    src4d = edge_index[0].astype(jnp.int32).reshape(NC * NS, _NB, _NSUB, SUB)
    dst4d = edge_index[1].astype(jnp.int32).reshape(NC * NS, _NB, _NSUB, SUB)
    dst3d = dst4d.reshape(NC * NS, _DEG_BLK, DSUB)
    wx = W[:D_FEAT]
    we = W[D_FEAT:]
    ones_sub = jnp.ones((DSUB,), f32)
    zeros_row = jnp.zeros((N_PAD // NS,), f32)
    zeros_acc = jnp.zeros((N_PAD // NS, 2 * D_OUT), f32)
    x_pad = jnp.pad(x, ((0, N_PAD - N_NODES), (0, 0)))

    mesh = plsc.VectorSubcoreMesh(core_axis_name="c", subcore_axis_name="s",
                                  num_cores=NC, num_subcores=NS)

    deg_fn = pl.kernel(
        _deg_body,
        out_type=jax.ShapeDtypeStruct((NC, N_PAD), f32),
        mesh=mesh,
        scratch_types=[
            pltpu.VMEM_SHARED((N_PAD,), f32),
            pltpu.VMEM((_DEG_BLK, DSUB), jnp.int32),
            pltpu.VMEM((DSUB,), f32),
            pltpu.SemaphoreType.DMA,
        ],
    )
    deg = deg_fn(dst3d, ones_sub, zeros_row)

    grid_n = N_PAD // 512
    yd = pl.pallas_call(
        _prep_body,
        grid=(grid_n,),
        in_specs=[
            pl.BlockSpec((512, 1), lambda i: (i, 0)),
            pl.BlockSpec((512, 1), lambda i: (i, 0)),
            pl.BlockSpec((512, D_FEAT), lambda i: (i, 0)),
            pl.BlockSpec((D_FEAT, D_OUT), lambda i: (0, 0)),
        ],
        out_specs=pl.BlockSpec((512, 2 * D_OUT), lambda i: (i, 0)),
        out_shape=jax.ShapeDtypeStruct((N_PAD, 2 * D_OUT), f32),
    )(deg[0].reshape(N_PAD, 1), deg[1].reshape(N_PAD, 1), x_pad, wx)

    z = pl.pallas_call(
        _edge_mm_body,
        grid=(N_EDGES // 2000,),
        in_specs=[
            pl.BlockSpec((2000, D_EDGE), lambda i: (i, 0)),
            pl.BlockSpec((D_EDGE, D_OUT), lambda i: (0, 0)),
        ],
        out_specs=pl.BlockSpec((2000, D_OUT), lambda i: (i, 0)),
        out_shape=jax.ShapeDtypeStruct((N_EDGES, D_OUT), f32),
    )(edge_attr, we)

    agg_fn = pl.kernel(
        _agg_body,
        out_type=jax.ShapeDtypeStruct((NC, N_PAD, 2 * D_OUT), f32),
        mesh=mesh,
        scratch_types=[
            pltpu.VMEM_SHARED((N_PAD, 2 * D_OUT), f32),
        ] + [
            t
            for _ in range(4)
            for t in (
                pltpu.VMEM((_NSUB, SUB), jnp.int32),
                pltpu.VMEM((_NSUB, SUB), jnp.int32),
                pltpu.VMEM((B_E, 2 * D_OUT), f32),
                pltpu.VMEM((B_E, D_OUT), f32),
                pltpu.SemaphoreType.DMA,
                pltpu.SemaphoreType.DMA,
                pltpu.SemaphoreType.DMA,
                pltpu.SemaphoreType.DMA,
            )
        ],
    )
    acc = agg_fn(src4d, dst4d, yd, z, zeros_acc)

    out = pl.pallas_call(
        _final_body,
        grid=(grid_n,),
        in_specs=[
            pl.BlockSpec((512, 2 * D_OUT), lambda i: (i, 0)),
            pl.BlockSpec((512, 2 * D_OUT), lambda i: (i, 0)),
            pl.BlockSpec((512, 2 * D_OUT), lambda i: (i, 0)),
            pl.BlockSpec((512, 1), lambda i: (i, 0)),
            pl.BlockSpec((512, 1), lambda i: (i, 0)),
            pl.BlockSpec((1, D_OUT), lambda i: (0, 0)),
        ],
        out_specs=pl.BlockSpec((512, D_OUT), lambda i: (i, 0)),
        out_shape=jax.ShapeDtypeStruct((N_PAD, D_OUT), f32),
    )(acc[0], acc[1], yd, deg[0].reshape(N_PAD, 1), deg[1].reshape(N_PAD, 1),
      b.reshape(1, D_OUT))

    return out[:N_NODES]


# speedup vs baseline: 1.1942x; 1.1942x over previous
import functools

import jax
import jax.numpy as jnp
from jax import lax
from jax.experimental import pallas as pl
from jax.experimental.pallas import tpu as pltpu
from jax.experimental.pallas import tpu_sc as plsc

N_NODES = 10000
N_PAD = 10240
N_EDGES = 320000
D_FEAT = 128
D_EDGE = 16
D_OUT = 64

NC = 2
NS = 16
EC = N_EDGES // (NC * NS)
SUB = 80
DSUB = 80


_DEG_BLK = EC // DSUB


def _deg_body(dst_hbm, ones_hbm, zeros_hbm, deg_out, deg_sp, dst_v, ones_v, sem):
    c = lax.axis_index("c")
    s = lax.axis_index("s")
    nrow = N_PAD // NS
    pltpu.sync_copy(zeros_hbm, deg_sp.at[pl.ds(s * nrow, nrow)])
    pltpu.sync_copy(ones_hbm, ones_v)
    pltpu.sync_copy(dst_hbm.at[c * NS + s], dst_v)
    plsc.subcore_barrier()

    def fire(j, _):
        pltpu.async_copy(ones_v, deg_sp.at[dst_v.at[j]], sem, add=True)
        return 0

    def drain(j, _):
        pltpu.make_async_copy(ones_v, deg_sp.at[dst_v.at[0]], sem).wait()
        return 0

    lax.fori_loop(0, _DEG_BLK, fire, 0)
    lax.fori_loop(0, _DEG_BLK, drain, 0)
    plsc.subcore_barrier()
    pltpu.sync_copy(deg_sp.at[pl.ds(s * nrow, nrow)],
                    deg_out.at[c, pl.ds(s * nrow, nrow)])


B_E = 80
_NB = EC // B_E
_NSUB = B_E // SUB


def _agg_body(sd_hbm, yd_hbm, z_hbm, zeros_hbm, acc_out,
              acc_sp, sd_a, yd_a, z_a, sd_b, yd_b, z_b,
              gs_a, zs_a, ss_a, gs_b, zs_b, ss_b):
    c = lax.axis_index("c")
    s = lax.axis_index("s")
    nrow = N_PAD // NS
    pltpu.sync_copy(zeros_hbm, acc_sp.at[pl.ds(s * nrow, nrow)])
    plsc.subcore_barrier()
    w = c * NS + s

    def start(i, sd_v, yd_v, z_v, gsem, zsem):
        pltpu.sync_copy(sd_hbm.at[w, i], sd_v)
        pltpu.async_copy(yd_hbm.at[sd_v.at[0]], yd_v, gsem)
        pltpu.async_copy(z_hbm.at[pl.ds((w * _NB + i) * B_E, B_E)], z_v, zsem)

    def finish(sd_v, yd_v, z_v, gsem, zsem, ssem):
        pltpu.make_async_copy(yd_hbm.at[sd_v.at[0]], yd_v, gsem).wait()
        pltpu.make_async_copy(z_hbm.at[pl.ds(0, B_E)], z_v, zsem).wait()

        def group(r, _):
            db = yd_v[r, pl.ds(D_OUT, 16)]
            for seg in range(D_OUT // 16):
                yv = yd_v[r, pl.ds(seg * 16, 16)]
                zv = z_v[r, pl.ds(seg * 16, 16)]
                yd_v[r, pl.ds(seg * 16, 16)] = yv + db * zv
            return 0

        lax.fori_loop(0, B_E, group, 0)
        pltpu.async_copy(yd_v, acc_sp.at[sd_v.at[1]], ssem, add=True)

    def drain(sd_v, yd_v, ssem):
        pltpu.make_async_copy(yd_v, acc_sp.at[sd_v.at[1]], ssem).wait()

    A = (sd_a, yd_a, z_a, gs_a, zs_a)
    B = (sd_b, yd_b, z_b, gs_b, zs_b)
    start(0, *A)
    start(1, *B)

    def pair(k, _):
        finish(*A, ss_a)
        @pl.when(2 * k + 2 < _NB)
        def _():
            drain(sd_a, yd_a, ss_a)
            start(2 * k + 2, *A)
        finish(*B, ss_b)
        @pl.when(2 * k + 3 < _NB)
        def _():
            drain(sd_b, yd_b, ss_b)
            start(2 * k + 3, *B)
        return 0

    lax.fori_loop(0, (_NB - 1) // 2, pair, 0)
    finish(*A, ss_a)
    drain(sd_a, yd_a, ss_a)
    drain(sd_b, yd_b, ss_b)
    plsc.subcore_barrier()
    pltpu.sync_copy(acc_sp.at[pl.ds(s * nrow, nrow)],
                    acc_out.at[c, pl.ds(s * nrow, nrow), :])


def _prep_body(deg0_ref, deg1_ref, x_ref, wx_ref, yd_ref):
    deg = deg0_ref[...] + deg1_ref[...] + 1.0
    dinv = deg ** -0.5
    y = jnp.dot(x_ref[...], wx_ref[...], preferred_element_type=jnp.float32)
    yd_ref[...] = jnp.concatenate(
        [dinv * y, jnp.broadcast_to(dinv, (dinv.shape[0], D_OUT))], axis=1)


def _edge_mm_body(ea_ref, we_ref, z_ref):
    z_ref[...] = jnp.dot(ea_ref[...], we_ref[...],
                         preferred_element_type=jnp.float32)


def _final_body(acc0_ref, acc1_ref, yd_ref, deg0_ref, deg1_ref,
                b_ref, out_ref):
    dinv = yd_ref[:, D_OUT:D_OUT + 1]
    ys = yd_ref[:, :D_OUT]
    acc = acc0_ref[:, :D_OUT] + acc1_ref[:, :D_OUT]
    deg = deg0_ref[...] + deg1_ref[...] + 1.0
    out_ref[...] = dinv * (acc + ys) + deg * b_ref[...]


@jax.jit
def kernel(x, edge_index, edge_attr, W, b):
    f32 = jnp.float32
    ei32 = edge_index.astype(jnp.int32)
    src4d = ei32[0].reshape(NC * NS, _NB, 1, SUB)
    dst4d = ei32[1].reshape(NC * NS, _NB, 1, SUB)
    sd4d = jnp.concatenate([src4d, dst4d], axis=2)
    dst3d = ei32[1].reshape(NC * NS, _DEG_BLK, DSUB)
    wx = W[:D_FEAT]
    we = W[D_FEAT:]
    ones_sub = jnp.ones((DSUB,), f32)
    zeros_row = jnp.zeros((N_PAD // NS,), f32)
    zeros_acc = jnp.zeros((N_PAD // NS, 2 * D_OUT), f32)
    x_pad = jnp.pad(x, ((0, N_PAD - N_NODES), (0, 0)))

    mesh = plsc.VectorSubcoreMesh(core_axis_name="c", subcore_axis_name="s",
                                  num_cores=NC, num_subcores=NS)

    deg_fn = pl.kernel(
        _deg_body,
        out_type=jax.ShapeDtypeStruct((NC, N_PAD), f32),
        mesh=mesh,
        scratch_types=[
            pltpu.VMEM_SHARED((N_PAD,), f32),
            pltpu.VMEM((_DEG_BLK, DSUB), jnp.int32),
            pltpu.VMEM((DSUB,), f32),
            pltpu.SemaphoreType.DMA,
        ],
    )
    deg = deg_fn(dst3d, ones_sub, zeros_row)

    grid_n = N_PAD // 512
    yd = pl.pallas_call(
        _prep_body,
        grid=(grid_n,),
        in_specs=[
            pl.BlockSpec((512, 1), lambda i: (i, 0)),
            pl.BlockSpec((512, 1), lambda i: (i, 0)),
            pl.BlockSpec((512, D_FEAT), lambda i: (i, 0)),
            pl.BlockSpec((D_FEAT, D_OUT), lambda i: (0, 0)),
        ],
        out_specs=pl.BlockSpec((512, 2 * D_OUT), lambda i: (i, 0)),
        out_shape=jax.ShapeDtypeStruct((N_PAD, 2 * D_OUT), f32),
    )(deg[0].reshape(N_PAD, 1), deg[1].reshape(N_PAD, 1), x_pad, wx)

    z = pl.pallas_call(
        _edge_mm_body,
        grid=(N_EDGES // 2000,),
        in_specs=[
            pl.BlockSpec((2000, D_EDGE), lambda i: (i, 0)),
            pl.BlockSpec((D_EDGE, D_OUT), lambda i: (0, 0)),
        ],
        out_specs=pl.BlockSpec((2000, D_OUT), lambda i: (i, 0)),
        out_shape=jax.ShapeDtypeStruct((N_EDGES, D_OUT), f32),
    )(edge_attr, we)

    agg_fn = pl.kernel(
        _agg_body,
        out_type=jax.ShapeDtypeStruct((NC, N_PAD, 2 * D_OUT), f32),
        mesh=mesh,
        scratch_types=[
            pltpu.VMEM_SHARED((N_PAD, 2 * D_OUT), f32),
            pltpu.VMEM((2, SUB), jnp.int32),
            pltpu.VMEM((B_E, 2 * D_OUT), f32),
            pltpu.VMEM((B_E, D_OUT), f32),
            pltpu.VMEM((2, SUB), jnp.int32),
            pltpu.VMEM((B_E, 2 * D_OUT), f32),
            pltpu.VMEM((B_E, D_OUT), f32),
            pltpu.SemaphoreType.DMA,
            pltpu.SemaphoreType.DMA,
            pltpu.SemaphoreType.DMA,
            pltpu.SemaphoreType.DMA,
            pltpu.SemaphoreType.DMA,
            pltpu.SemaphoreType.DMA,
        ],
    )
    acc = agg_fn(sd4d, yd, z, zeros_acc)

    out = pl.pallas_call(
        _final_body,
        grid=(grid_n,),
        in_specs=[
            pl.BlockSpec((512, 2 * D_OUT), lambda i: (i, 0)),
            pl.BlockSpec((512, 2 * D_OUT), lambda i: (i, 0)),
            pl.BlockSpec((512, 2 * D_OUT), lambda i: (i, 0)),
            pl.BlockSpec((512, 1), lambda i: (i, 0)),
            pl.BlockSpec((512, 1), lambda i: (i, 0)),
            pl.BlockSpec((1, D_OUT), lambda i: (0, 0)),
        ],
        out_specs=pl.BlockSpec((512, D_OUT), lambda i: (i, 0)),
        out_shape=jax.ShapeDtypeStruct((N_PAD, D_OUT), f32),
    )(acc[0], acc[1], yd, deg[0].reshape(N_PAD, 1), deg[1].reshape(N_PAD, 1),
      b.reshape(1, D_OUT))

    return out[:N_NODES]
